# Initial kernel scaffold; baseline (speedup 1.0000x reference)
import jax
import jax.numpy as jnp
from jax.experimental import pallas as pl


def kernel(x, embed_weight):
    raise NotImplementedError("write your pallas kernel here")


# speedup vs baseline: 1.5479x; 1.5479x over previous
import functools

import jax
import jax.numpy as jnp
from jax import lax
from jax.experimental import pallas as pl
from jax.experimental.pallas import tpu as pltpu
from jax.experimental.pallas import tpu_sc as plsc

D_MODEL = 32
_ROWS, _COLS = 16384, 26
_B = _ROWS * _COLS
_L = 128
_NCHUNK = 8
_SC = _NCHUNK * _L
_NW = 32
_B_PER_W = _B // _NW
_NSC = _B_PER_W // _SC
_IDX_ROWS_PER_W = _B_PER_W // _L


def _make_gather():
    mesh = plsc.VectorSubcoreMesh(core_axis_name="c", subcore_axis_name="s")

    @functools.partial(
        pl.kernel,
        out_type=jax.ShapeDtypeStruct((_B, D_MODEL), jnp.float32),
        mesh=mesh,
        scratch_types=[
            pltpu.VMEM((_NCHUNK, _L), jnp.int32),
            pltpu.VMEM((_SC, D_MODEL), jnp.float32),
            pltpu.SemaphoreType.DMA,
        ],
        compiler_params=pltpu.CompilerParams(use_tc_tiling_on_sc=False),
    )
    def gather(table_hbm, idx_hbm, out_hbm, idx_v, rows_v, sem):
        wid = lax.axis_index("s") * 2 + lax.axis_index("c")
        row_base = wid * _IDX_ROWS_PER_W
        out_base = wid * _B_PER_W

        @pl.loop(0, _NSC)
        def _superchunk(g):
            pltpu.sync_copy(idx_hbm.at[pl.ds(row_base + g * _NCHUNK, _NCHUNK)],
                            idx_v)
            copies = []
            for j in range(_NCHUNK):
                copies.append(
                    pltpu.async_copy(table_hbm.at[idx_v.at[j]],
                                     rows_v.at[pl.ds(j * _L, _L)], sem))
            for c in copies:
                c.wait()
            pltpu.sync_copy(rows_v,
                            out_hbm.at[pl.ds(out_base + g * _SC, _SC)])

    return gather


_gather = _make_gather()


@jax.jit
def kernel(x, embed_weight):
    idx = x.astype(jnp.int32).reshape(_B // _L, _L)
    out = _gather(embed_weight, idx)
    return out.reshape(_ROWS, _COLS, D_MODEL)


# speedup vs baseline: 1.5724x; 1.0159x over previous
import functools

import jax
import jax.numpy as jnp
from jax import lax
from jax.experimental import pallas as pl
from jax.experimental.pallas import tpu as pltpu
from jax.experimental.pallas import tpu_sc as plsc

D_MODEL = 32
_ROWS, _COLS = 16384, 26
_B = _ROWS * _COLS
_L = 128
_NCHUNK = 13
_SC = _NCHUNK * _L
_NW = 32
_B_PER_W = _B // _NW
_NSC = _B_PER_W // _SC
_IDX_ROWS_PER_W = _B_PER_W // _L


def _make_gather():
    mesh = plsc.VectorSubcoreMesh(core_axis_name="c", subcore_axis_name="s")

    @functools.partial(
        pl.kernel,
        out_type=jax.ShapeDtypeStruct((_B, D_MODEL), jnp.float32),
        mesh=mesh,
        scratch_types=[
            pltpu.VMEM((2, _NCHUNK, _L), jnp.int32),
            pltpu.VMEM((2, _SC, D_MODEL), jnp.float32),
            pltpu.SemaphoreType.DMA,
            pltpu.SemaphoreType.DMA,
            pltpu.SemaphoreType.DMA,
            pltpu.SemaphoreType.DMA,
        ],
        compiler_params=pltpu.CompilerParams(use_tc_tiling_on_sc=False),
    )
    def gather(table_hbm, idx_hbm, out_hbm, idx_v, rows_v, gsem0, gsem1,
               osem0, osem1):
        wid = lax.axis_index("s") * 2 + lax.axis_index("c")
        row_base = wid * _IDX_ROWS_PER_W
        out_base = wid * _B_PER_W
        gsems = (gsem0, gsem1)
        osems = (osem0, osem1)

        @pl.loop(0, _NSC, step=2)
        def _pair(go):
            for b in range(2):
                g = go + b

                @pl.when(go != 0)
                def _reclaim():
                    pltpu.make_async_copy(
                        rows_v.at[b],
                        out_hbm.at[pl.ds(out_base + g * _SC, _SC)],
                        osems[b]).wait()

                pltpu.sync_copy(
                    idx_hbm.at[pl.ds(row_base + g * _NCHUNK, _NCHUNK)],
                    idx_v.at[b])
                for j in range(_NCHUNK):
                    pltpu.async_copy(table_hbm.at[idx_v.at[b, j]],
                                     rows_v.at[b, pl.ds(j * _L, _L)],
                                     gsems[b])
            for b in range(2):
                g = go + b
                for j in range(_NCHUNK):
                    pltpu.make_async_copy(table_hbm.at[idx_v.at[b, j]],
                                          rows_v.at[b, pl.ds(j * _L, _L)],
                                          gsems[b]).wait()
                pltpu.async_copy(rows_v.at[b],
                                 out_hbm.at[pl.ds(out_base + g * _SC, _SC)],
                                 osems[b])

        for b in range(2):
            pltpu.make_async_copy(
                rows_v.at[b],
                out_hbm.at[pl.ds(out_base + (_NSC - 2 + b) * _SC, _SC)],
                osems[b]).wait()

    return gather


_gather = _make_gather()


@jax.jit
def kernel(x, embed_weight):
    idx = x.astype(jnp.int32).reshape(_B // _L, _L)
    out = _gather(embed_weight, idx)
    return out.reshape(_ROWS, _COLS, D_MODEL)


# speedup vs baseline: 1.6693x; 1.0616x over previous
import functools

import jax
import jax.numpy as jnp
from jax import lax
from jax.experimental import pallas as pl
from jax.experimental.pallas import tpu as pltpu
from jax.experimental.pallas import tpu_sc as plsc

D_MODEL = 32
_ROWS, _COLS = 16384, 26
_B = _ROWS * _COLS
_L = 128
_NCHUNK = 13
_SC = _NCHUNK * _L
_NW = 32
_B_PER_W = _B // _NW
_NSC = _B_PER_W // _SC


def _make_gather():
    mesh = plsc.VectorSubcoreMesh(core_axis_name="c", subcore_axis_name="s")

    @functools.partial(
        pl.kernel,
        out_type=jax.ShapeDtypeStruct((_B, D_MODEL), jnp.float32),
        mesh=mesh,
        scratch_types=[
            pltpu.VMEM((2, _SC), jnp.int32),
            pltpu.VMEM((2, _SC, D_MODEL), jnp.float32),
            pltpu.SemaphoreType.DMA,
            pltpu.SemaphoreType.DMA,
            pltpu.SemaphoreType.DMA,
            pltpu.SemaphoreType.DMA,
        ],
        compiler_params=pltpu.CompilerParams(use_tc_tiling_on_sc=False),
    )
    def gather(table_hbm, idx_hbm, out_hbm, idx_v, rows_v, gsem0, gsem1,
               osem0, osem1):
        wid = lax.axis_index("s") * 2 + lax.axis_index("c")
        base = wid * _B_PER_W
        gsems = (gsem0, gsem1)
        osems = (osem0, osem1)

        @pl.loop(0, _NSC, step=2)
        def _pair(go):
            for b in range(2):
                g = go + b

                @pl.when(go != 0)
                def _reclaim():
                    pltpu.make_async_copy(
                        rows_v.at[b],
                        out_hbm.at[pl.ds(base + g * _SC, _SC)],
                        osems[b]).wait()

                pltpu.sync_copy(idx_hbm.at[pl.ds(base + g * _SC, _SC)],
                                idx_v.at[b])
                for j in range(_NCHUNK):
                    pltpu.async_copy(
                        table_hbm.at[idx_v.at[b, pl.ds(j * _L, _L)]],
                        rows_v.at[b, pl.ds(j * _L, _L)],
                        gsems[b])
            for b in range(2):
                g = go + b
                for j in range(_NCHUNK):
                    pltpu.make_async_copy(
                        table_hbm.at[idx_v.at[b, pl.ds(j * _L, _L)]],
                        rows_v.at[b, pl.ds(j * _L, _L)],
                        gsems[b]).wait()
                pltpu.async_copy(rows_v.at[b],
                                 out_hbm.at[pl.ds(base + g * _SC, _SC)],
                                 osems[b])

        for b in range(2):
            pltpu.make_async_copy(
                rows_v.at[b],
                out_hbm.at[pl.ds(base + (_NSC - 2 + b) * _SC, _SC)],
                osems[b]).wait()

    return gather


_gather = _make_gather()


@jax.jit
def kernel(x, embed_weight):
    idx = jnp.transpose(x).reshape(_B).astype(jnp.int32)
    out = _gather(embed_weight, idx)
    return jnp.transpose(out.reshape(_COLS, _ROWS, D_MODEL), (1, 0, 2))
